# baseline (speedup 1.0000x reference)
import functools

import jax
import jax.numpy as jnp
from jax import lax
from jax.experimental import pallas as pl
from jax.experimental.pallas import tpu as pltpu
from jax.experimental.pallas import tpu_sc as plsc

EMB = 16
N_NODES = 100000
N_EDGES = 3200000

NC = 2
NS = 16
NW = NC * NS

NPAD = 100352
N8 = NPAD // 8
TSLICE = NPAD // NS

CHUNK_P = 512
CHUNK_D = 2048
E_PAD = NW * CHUNK_D * (-(-N_EDGES // (NW * CHUNK_D)))
EW = E_PAD // NW


def _gather_rows(table_hbm, idx_ref, dst_ref, sem):
    return pltpu.async_copy(table_hbm.at[idx_ref], dst_ref, sem)


def _scatter_add(src_ref, acc_ref, idx_ref):
    pltpu.sync_copy(src_ref, acc_ref.at[idx_ref], add=True)


def _gather16(ref, idx0):
    return plsc.load_gather(ref, [idx0])


def _edge_loop(n, fn, unroll):
    plsc.parallel_loop(0, n, 1, unroll=unroll)(fn)


def _axis_index(name):
    return lax.axis_index(name)


def _barrier():
    plsc.subcore_barrier()


def _sc_mesh():
    return plsc.VectorSubcoreMesh(core_axis_name="c", subcore_axis_name="s",
                                  num_cores=NC, num_subcores=NS)


_SC_PARAMS = dict(
    compiler_params=pltpu.CompilerParams(
        needs_layout_passes=False, use_tc_tiling_on_sc=False))


def _build_sc_deg():
    sub = CHUNK_D // 128
    g_n = EW // CHUNK_D
    out_type = [jax.ShapeDtypeStruct((NPAD,), jnp.float32)
                for _ in range(2 * NC)]
    scratch = [
        pltpu.VMEM_SHARED((NPAD,), jnp.float32),
        pltpu.VMEM_SHARED((NPAD,), jnp.float32),
        pltpu.VMEM((sub, 128), jnp.int32),
        pltpu.VMEM((sub, 128), jnp.int32),
        pltpu.VMEM((128,), jnp.float32),
    ]

    def body(aidx_hbm, bidx_hbm, z1d_hbm, *rest):
        dega_outs = rest[:NC]
        degb_outs = rest[NC:2 * NC]
        dega_sh, degb_sh, aidx_v, bidx_v, ones_v = rest[2 * NC:]

        cid = _axis_index("c")
        sid = _axis_index("s")
        wid = cid * NS + sid
        toff = sid * TSLICE

        pltpu.sync_copy(z1d_hbm, dega_sh.at[pl.ds(toff, TSLICE)])
        pltpu.sync_copy(z1d_hbm, degb_sh.at[pl.ds(toff, TSLICE)])
        ones16 = jnp.full((16,), 1.0, jnp.float32)
        for t in range(8):
            ones_v[pl.ds(16 * t, 16)] = ones16
        _barrier()

        row0 = wid * (EW // 128)

        def super_body(g, carry):
            rb = row0 + g * sub
            pltpu.sync_copy(aidx_hbm.at[pl.ds(rb, sub), :], aidx_v)
            pltpu.sync_copy(bidx_hbm.at[pl.ds(rb, sub), :], bidx_v)
            for i in range(sub):
                _scatter_add(ones_v, dega_sh, aidx_v.at[i])
                _scatter_add(ones_v, degb_sh, bidx_v.at[i])
            return carry

        lax.fori_loop(0, g_n, super_body, 0)
        _barrier()

        for k in range(NC):
            @pl.when(cid == k)
            def _copy_degs(k=k):
                pltpu.sync_copy(dega_sh.at[pl.ds(toff, TSLICE)],
                                dega_outs[k].at[pl.ds(toff, TSLICE)])
                pltpu.sync_copy(degb_sh.at[pl.ds(toff, TSLICE)],
                                degb_outs[k].at[pl.ds(toff, TSLICE)])

    return pl.kernel(body, out_type=out_type, mesh=_sc_mesh(),
                     scratch_types=scratch, **_SC_PARAMS)


def _build_sc_pass():
    sub = CHUNK_P // 128
    g_n = EW // CHUNK_P
    out_type = [jax.ShapeDtypeStruct((NC, NPAD, EMB), jnp.float32)]
    scratch = [
        pltpu.VMEM_SHARED((NPAD, EMB), jnp.float32),
        pltpu.VMEM((sub, 128), jnp.int32),
        pltpu.VMEM((sub, 128), jnp.int32),
        pltpu.VMEM((CHUNK_P,), jnp.float32),
        pltpu.VMEM((CHUNK_P, EMB), jnp.float32),
        pltpu.VMEM((CHUNK_P, EMB), jnp.float32),
        pltpu.VMEM((4, EMB), jnp.float32),
        pltpu.SemaphoreType.DMA,
    ]

    def body(aidx_hbm, bidx_hbm, ef_hbm, left_hbm, right_hbm, w_hbm,
             z2d_hbm, s_out, s_sh, aidx_v, bidx_v, ef_v, l_v, r_v, w_v, sem):
        cid = _axis_index("c")
        sid = _axis_index("s")
        wid = cid * NS + sid
        toff = sid * TSLICE

        pltpu.sync_copy(z2d_hbm, s_sh.at[pl.ds(toff, TSLICE), :])
        pltpu.sync_copy(w_hbm, w_v)
        _barrier()

        fe = w_v[0]
        eew = w_v[1]
        eeb = w_v[2]
        zero16 = jnp.zeros((16,), jnp.float32)

        row0 = wid * (EW // 128)

        def super_body(g, carry):
            rb = row0 + g * sub
            eb = wid * EW + g * CHUNK_P
            pltpu.sync_copy(aidx_hbm.at[pl.ds(rb, sub), :], aidx_v)
            pltpu.sync_copy(bidx_hbm.at[pl.ds(rb, sub), :], bidx_v)
            pltpu.sync_copy(ef_hbm.at[pl.ds(eb, CHUNK_P)], ef_v)
            descs = []
            for i in range(sub):
                descs.append(_gather_rows(
                    left_hbm, aidx_v.at[i],
                    l_v.at[pl.ds(i * 128, 128), :], sem))
            for i in range(sub):
                descs.append(_gather_rows(
                    right_hbm, bidx_v.at[i],
                    r_v.at[pl.ds(i * 128, 128), :], sem))
            for t in range(CHUNK_P // 16):
                s = pl.ds(16 * t, 16)
                ef_v[s] = jnp.maximum(ef_v[s] * eew + eeb, zero16)
            for d in descs:
                d.wait()

            def edge_body(j):
                bj = jnp.broadcast_to(j, (16,))
                e16 = _gather16(ef_v, bj)
                l_v[j] = jnp.maximum(l_v[j] + r_v[j] + e16 * fe, zero16)
            _edge_loop(CHUNK_P, edge_body, 16)

            for i in range(sub):
                _scatter_add(l_v.at[pl.ds(i * 128, 128), :],
                             s_sh, bidx_v.at[i])
            return carry

        lax.fori_loop(0, g_n, super_body, 0)
        _barrier()

        pltpu.sync_copy(s_sh.at[pl.ds(toff, TSLICE), :],
                        s_out.at[cid, pl.ds(toff, TSLICE), :])

    return pl.kernel(body, out_type=out_type, mesh=_sc_mesh(),
                     scratch_types=scratch, **_SC_PARAMS)


BR8 = 3136


def _kron8(w):
    return jnp.kron(jnp.eye(8, dtype=jnp.float32), w.astype(jnp.float32))


def _tile8(b):
    return jnp.tile(b.astype(jnp.float32), 8).reshape(1, 128)


def _dot(x, w):
    return jnp.dot(x, w, preferred_element_type=jnp.float32)


def _tc_embed_kernel(cf_ref, vf_ref, ce1, ceb1, ce2, ceb2, ve1, veb1, ve2,
                     veb2, fl1, flb1, fr1, fl2, flb2,
                     c0_ref, rt1_ref, v0_ref, lt1_ref, rt2_ref):
    c = jnp.maximum(_dot(cf_ref[...], ce1[...]) + ceb1[...], 0.0)
    c = jnp.maximum(_dot(c, ce2[...]) + ceb2[...], 0.0)
    c0_ref[...] = c
    rt1_ref[...] = _dot(c, fl1[...]) + flb1[...]
    v = jnp.maximum(_dot(vf_ref[...], ve1[...]) + veb1[...], 0.0)
    v = jnp.maximum(_dot(v, ve2[...]) + veb2[...], 0.0)
    v0_ref[...] = v
    lt1_ref[...] = _dot(v, fr1[...])
    rt2_ref[...] = _dot(v, fl2[...]) + flb2[...]


def _tc_post_kernel(s_ref, deg_ref, right_ref, rep, ff, ffb, pc, pcb, o1a,
                    o1b, o1bias, o2, o2b, fr_next, outw, outb,
                    lt_ref, oo_ref):
    s = jnp.sum(s_ref[...], axis=0)
    deg = jnp.sum(deg_ref[...], axis=0)
    agg = _dot(s, ff[...]) + _dot(deg, rep[...]) * ffb[...]
    h = _dot(jnp.maximum(agg, 0.0), pc[...]) + pcb[...]
    right = right_ref[...]
    h2 = jnp.maximum(_dot(h, o1a[...]) + _dot(right, o1b[...]) + o1bias[...],
                     0.0)
    node = _dot(h2, o2[...]) + o2b[...]
    if lt_ref is not None:
        lt_ref[...] = _dot(node, fr_next[...])
    oo_ref[...] = jnp.maximum(_dot(node, outw[...]) + outb[...], 0.0)


def _run_embed(cfp, vfp, mats):
    grid = N8 // BR8
    row_spec = pl.BlockSpec((BR8, 128), lambda i: (i, 0))
    mat_specs = [pl.BlockSpec(m.shape, lambda i: (0, 0)) for m in mats]
    out_sds = jax.ShapeDtypeStruct((N8, 128), jnp.float32)
    return pl.pallas_call(
        _tc_embed_kernel,
        grid=(grid,),
        in_specs=[row_spec, row_spec] + mat_specs,
        out_specs=[row_spec] * 5,
        out_shape=[out_sds] * 5,
    )(cfp, vfp, *mats)


def _run_post(s_parts, deg_parts, rightp, mats, with_lt):
    grid = N8 // BR8
    s_spec = pl.BlockSpec((NC, BR8, 128), lambda i: (0, i, 0))
    deg_spec = pl.BlockSpec((NC, BR8, 8), lambda i: (0, i, 0))
    row_spec = pl.BlockSpec((BR8, 128), lambda i: (i, 0))
    mat_specs = [pl.BlockSpec(m.shape, lambda i: (0, 0)) for m in mats]
    out_sds = jax.ShapeDtypeStruct((N8, 128), jnp.float32)
    if with_lt:
        kern = lambda *a: _tc_post_kernel(*a[:16], a[16], a[17])
        out_specs, out_shape = [row_spec] * 2, [out_sds] * 2
    else:
        kern = lambda *a: _tc_post_kernel(*a[:13], None, a[13], a[14], None,
                                          a[15])
        out_specs, out_shape = [row_spec], [out_sds]
    return pl.pallas_call(
        kern,
        grid=(grid,),
        in_specs=[s_spec, deg_spec, row_spec] + mat_specs,
        out_specs=out_specs,
        out_shape=out_shape,
    )(s_parts, deg_parts, rightp, *mats)


def kernel(constraint_features, edge_indices, edge_features, variable_features,
           params):
    p = params
    f32 = jnp.float32

    def pack_nodes(x):
        x = jnp.pad(x.astype(f32), ((0, NPAD - N_NODES), (0, 0)))
        return x.reshape(N8, 128)

    cfp = pack_nodes(constraint_features)
    vfp = pack_nodes(variable_features)

    e0 = edge_indices[0].astype(jnp.int32)
    e1 = edge_indices[1].astype(jnp.int32)
    pad_e = E_PAD - N_EDGES
    e0 = jnp.pad(e0, (0, pad_e), constant_values=N_NODES).reshape(-1, 128)
    e1 = jnp.pad(e1, (0, pad_e), constant_values=N_NODES).reshape(-1, 128)
    ef = jnp.pad(edge_features.astype(f32).reshape(-1), (0, pad_e))

    z2d = jnp.zeros((TSLICE, EMB), f32)
    z1d = jnp.zeros((TSLICE,), f32)

    def wrow(fe_w, eew, eeb):
        return jnp.stack([fe_w.astype(f32).reshape(EMB),
                          jnp.full((EMB,), eew, f32),
                          jnp.full((EMB,), eeb, f32),
                          jnp.zeros((EMB,), f32)])

    eew = p['ee_w'].reshape(())
    eeb = p['ee_b'].reshape(())
    w1 = wrow(p['cv']['fe_w'], eew, eeb)
    w2 = wrow(p['cc']['fe_w'], eew, eeb)

    rep = jnp.repeat(jnp.eye(8, dtype=f32), 16, axis=1)

    embed_mats = [
        _kron8(p['ce_w1']), _tile8(p['ce_b1']),
        _kron8(p['ce_w2']), _tile8(p['ce_b2']),
        _kron8(p['ve_w1']), _tile8(p['ve_b1']),
        _kron8(p['ve_w2']), _tile8(p['ve_b2']),
        _kron8(p['cv']['fl_w']), _tile8(p['cv']['fl_b']),
        _kron8(p['cv']['fr_w']),
        _kron8(p['cc']['fl_w']), _tile8(p['cc']['fl_b']),
    ]
    c0p, rt1p, v0p, lt1p, rt2p = _run_embed(cfp, vfp, embed_mats)

    deg_outs = _build_sc_deg()(e1, e0, z1d)
    dega = jnp.stack(deg_outs[:NC])
    degb = jnp.stack(deg_outs[NC:])
    deg1 = degb.reshape(NC, N8, 8)
    deg2 = dega.reshape(NC, N8, 8)

    sc_edge = _build_sc_pass()
    lt1 = lt1p.reshape(NPAD, EMB)
    rt1 = rt1p.reshape(NPAD, EMB)
    (s1_parts,) = sc_edge(e1, e0, ef, lt1, rt1, w1, z2d)
    s1 = s1_parts.reshape(NC, N8, 128)

    pcv = p['cv']
    post1_mats = [
        rep, _kron8(pcv['ff_w']), _tile8(pcv['ff_b']),
        _kron8(pcv['pc_w']), _tile8(pcv['pc_b']),
        _kron8(pcv['o1_w'][:EMB]), _kron8(pcv['o1_w'][EMB:]),
        _tile8(pcv['o1_b']),
        _kron8(pcv['o2_w']), _tile8(pcv['o2_b']),
        _kron8(p['cc']['fr_w']),
        _kron8(p['oc_w']), _tile8(p['oc_b']),
    ]
    lt2p, ocp = _run_post(s1, deg1, c0p, post1_mats, with_lt=True)

    lt2 = lt2p.reshape(NPAD, EMB)
    rt2 = rt2p.reshape(NPAD, EMB)
    (s2_parts,) = sc_edge(e0, e1, ef, lt2, rt2, w2, z2d)
    s2 = s2_parts.reshape(NC, N8, 128)

    pcc = p['cc']
    post2_mats = [
        rep, _kron8(pcc['ff_w']), _tile8(pcc['ff_b']),
        _kron8(pcc['pc_w']), _tile8(pcc['pc_b']),
        _kron8(pcc['o1_w'][:EMB]), _kron8(pcc['o1_w'][EMB:]),
        _tile8(pcc['o1_b']),
        _kron8(pcc['o2_w']), _tile8(pcc['o2_b']),
        _kron8(p['ov_w']), _tile8(p['ov_b']),
    ]
    (ovp,) = _run_post(s2, deg2, v0p, post2_mats, with_lt=False)

    out_var = ovp.reshape(NPAD, EMB)[:N_NODES]
    out_cons = ocp.reshape(NPAD, EMB)[:N_NODES]
    return (out_var, out_cons)


# speedup vs baseline: 1.0749x; 1.0749x over previous
import functools

import jax
import jax.numpy as jnp
from jax import lax
from jax.experimental import pallas as pl
from jax.experimental.pallas import tpu as pltpu
from jax.experimental.pallas import tpu_sc as plsc

EMB = 16
N_NODES = 100000
N_EDGES = 3200000

NC = 2
NS = 16
NW = NC * NS

NPAD = 100352
N8 = NPAD // 8
TSLICE = NPAD // NS

CHUNK_P = 256
CHUNK_D = 2048
E_PAD = NW * CHUNK_D * (-(-N_EDGES // (NW * CHUNK_D)))
EW = E_PAD // NW


def _gather_rows(table_hbm, idx_ref, dst_ref, sem):
    return pltpu.async_copy(table_hbm.at[idx_ref], dst_ref, sem)


def _gather_drain(table_hbm, idx_ref, dst_ref, sem):
    pltpu.make_async_copy(table_hbm.at[idx_ref], dst_ref, sem).wait()


def _scatter_add(src_ref, acc_ref, idx_ref):
    pltpu.sync_copy(src_ref, acc_ref.at[idx_ref], add=True)


def _gather16(ref, idx0):
    return plsc.load_gather(ref, [idx0])


def _edge_loop(n, fn, unroll):
    plsc.parallel_loop(0, n, 1, unroll=unroll)(fn)


def _axis_index(name):
    return lax.axis_index(name)


def _barrier():
    plsc.subcore_barrier()


def _sc_mesh():
    return plsc.VectorSubcoreMesh(core_axis_name="c", subcore_axis_name="s",
                                  num_cores=NC, num_subcores=NS)


_SC_PARAMS = dict(
    compiler_params=pltpu.CompilerParams(
        needs_layout_passes=False, use_tc_tiling_on_sc=False))


def _build_sc_deg():
    sub = CHUNK_D // 128
    g_n = EW // CHUNK_D
    out_type = [jax.ShapeDtypeStruct((NPAD,), jnp.float32)
                for _ in range(2 * NC)]
    scratch = [
        pltpu.VMEM_SHARED((NPAD,), jnp.float32),
        pltpu.VMEM_SHARED((NPAD,), jnp.float32),
        pltpu.VMEM((sub, 128), jnp.int32),
        pltpu.VMEM((sub, 128), jnp.int32),
        pltpu.VMEM((128,), jnp.float32),
    ]

    def body(aidx_hbm, bidx_hbm, z1d_hbm, *rest):
        dega_outs = rest[:NC]
        degb_outs = rest[NC:2 * NC]
        dega_sh, degb_sh, aidx_v, bidx_v, ones_v = rest[2 * NC:]

        cid = _axis_index("c")
        sid = _axis_index("s")
        wid = cid * NS + sid
        toff = sid * TSLICE

        pltpu.sync_copy(z1d_hbm, dega_sh.at[pl.ds(toff, TSLICE)])
        pltpu.sync_copy(z1d_hbm, degb_sh.at[pl.ds(toff, TSLICE)])
        ones16 = jnp.full((16,), 1.0, jnp.float32)
        for t in range(8):
            ones_v[pl.ds(16 * t, 16)] = ones16
        _barrier()

        row0 = wid * (EW // 128)

        def super_body(g, carry):
            rb = row0 + g * sub
            pltpu.sync_copy(aidx_hbm.at[pl.ds(rb, sub), :], aidx_v)
            pltpu.sync_copy(bidx_hbm.at[pl.ds(rb, sub), :], bidx_v)
            for i in range(sub):
                _scatter_add(ones_v, dega_sh, aidx_v.at[i])
                _scatter_add(ones_v, degb_sh, bidx_v.at[i])
            return carry

        lax.fori_loop(0, g_n, super_body, 0)
        _barrier()

        for k in range(NC):
            @pl.when(cid == k)
            def _copy_degs(k=k):
                pltpu.sync_copy(dega_sh.at[pl.ds(toff, TSLICE)],
                                dega_outs[k].at[pl.ds(toff, TSLICE)])
                pltpu.sync_copy(degb_sh.at[pl.ds(toff, TSLICE)],
                                degb_outs[k].at[pl.ds(toff, TSLICE)])

    return pl.kernel(body, out_type=out_type, mesh=_sc_mesh(),
                     scratch_types=scratch, **_SC_PARAMS)


def _build_sc_pass():
    sub = CHUNK_P // 128
    g_n = EW // CHUNK_P
    assert g_n % 2 == 0
    out_type = [jax.ShapeDtypeStruct((NC, NPAD, EMB), jnp.float32)]
    buf = lambda: [
        pltpu.VMEM((sub, 128), jnp.int32),
        pltpu.VMEM((sub, 128), jnp.int32),
        pltpu.VMEM((CHUNK_P,), jnp.float32),
        pltpu.VMEM((CHUNK_P, EMB), jnp.float32),
        pltpu.VMEM((CHUNK_P, EMB), jnp.float32),
        pltpu.SemaphoreType.DMA,
    ]
    scratch = (
        [pltpu.VMEM_SHARED((NPAD, EMB), jnp.float32)]
        + buf() + buf()
        + [pltpu.VMEM((4, EMB), jnp.float32)]
    )

    def body(aidx_hbm, bidx_hbm, ef_hbm, left_hbm, right_hbm, w_hbm,
             z2d_hbm, s_out, s_sh, *rest):
        bufs = [rest[0:6], rest[6:12]]
        w_v = rest[12]
        cid = _axis_index("c")
        sid = _axis_index("s")
        wid = cid * NS + sid
        toff = sid * TSLICE

        pltpu.sync_copy(z2d_hbm, s_sh.at[pl.ds(toff, TSLICE), :])
        pltpu.sync_copy(w_hbm, w_v)
        _barrier()

        fe = w_v[0]
        eew = w_v[1]
        eeb = w_v[2]
        zero16 = jnp.zeros((16,), jnp.float32)

        row0 = wid * (EW // 128)
        eb0 = wid * EW

        def load_and_fire(g, b):
            aidx_v, bidx_v, ef_v, l_v, r_v, sem = bufs[b]
            rb = row0 + g * sub
            eb = eb0 + g * CHUNK_P
            pltpu.sync_copy(aidx_hbm.at[pl.ds(rb, sub), :], aidx_v)
            pltpu.sync_copy(bidx_hbm.at[pl.ds(rb, sub), :], bidx_v)
            pltpu.sync_copy(ef_hbm.at[pl.ds(eb, CHUNK_P)], ef_v)
            for i in range(sub):
                _gather_rows(left_hbm, aidx_v.at[i],
                             l_v.at[pl.ds(i * 128, 128), :], sem)
                _gather_rows(right_hbm, bidx_v.at[i],
                             r_v.at[pl.ds(i * 128, 128), :], sem)
            for t in range(CHUNK_P // 16):
                s = pl.ds(16 * t, 16)
                ef_v[s] = jnp.maximum(ef_v[s] * eew + eeb, zero16)

        def drain_compute_scatter(b):
            aidx_v, bidx_v, ef_v, l_v, r_v, sem = bufs[b]
            for i in range(sub):
                _gather_drain(left_hbm, aidx_v.at[i],
                              l_v.at[pl.ds(i * 128, 128), :], sem)
                _gather_drain(right_hbm, bidx_v.at[i],
                              r_v.at[pl.ds(i * 128, 128), :], sem)

            def edge_body(j):
                bj = jnp.broadcast_to(j, (16,))
                e16 = _gather16(ef_v, bj)
                l_v[j] = jnp.maximum(l_v[j] + r_v[j] + e16 * fe, zero16)
            _edge_loop(CHUNK_P, edge_body, 8)

            for i in range(sub):
                _scatter_add(l_v.at[pl.ds(i * 128, 128), :],
                             s_sh, bidx_v.at[i])

        load_and_fire(0, 0)

        def super_body(h, carry):
            g = 2 * h
            load_and_fire(g + 1, 1)
            drain_compute_scatter(0)

            @pl.when(h < g_n // 2 - 1)
            def _prefetch():
                load_and_fire(g + 2, 0)
            drain_compute_scatter(1)
            return carry

        lax.fori_loop(0, g_n // 2, super_body, 0)
        _barrier()

        pltpu.sync_copy(s_sh.at[pl.ds(toff, TSLICE), :],
                        s_out.at[cid, pl.ds(toff, TSLICE), :])

    return pl.kernel(body, out_type=out_type, mesh=_sc_mesh(),
                     scratch_types=scratch, **_SC_PARAMS)


BR8 = 3136


def _kron8(w):
    return jnp.kron(jnp.eye(8, dtype=jnp.float32), w.astype(jnp.float32))


def _tile8(b):
    return jnp.tile(b.astype(jnp.float32), 8).reshape(1, 128)


def _dot(x, w):
    return jnp.dot(x, w, preferred_element_type=jnp.float32)


def _tc_embed_kernel(cf_ref, vf_ref, ce1, ceb1, ce2, ceb2, ve1, veb1, ve2,
                     veb2, fl1, flb1, fr1, fl2, flb2,
                     c0_ref, rt1_ref, v0_ref, lt1_ref, rt2_ref):
    c = jnp.maximum(_dot(cf_ref[...], ce1[...]) + ceb1[...], 0.0)
    c = jnp.maximum(_dot(c, ce2[...]) + ceb2[...], 0.0)
    c0_ref[...] = c
    rt1_ref[...] = _dot(c, fl1[...]) + flb1[...]
    v = jnp.maximum(_dot(vf_ref[...], ve1[...]) + veb1[...], 0.0)
    v = jnp.maximum(_dot(v, ve2[...]) + veb2[...], 0.0)
    v0_ref[...] = v
    lt1_ref[...] = _dot(v, fr1[...])
    rt2_ref[...] = _dot(v, fl2[...]) + flb2[...]


def _tc_post_kernel(s_ref, deg_ref, right_ref, rep, ff, ffb, pc, pcb, o1a,
                    o1b, o1bias, o2, o2b, fr_next, outw, outb,
                    lt_ref, oo_ref):
    s = jnp.sum(s_ref[...], axis=0)
    deg = jnp.sum(deg_ref[...], axis=0)
    agg = _dot(s, ff[...]) + _dot(deg, rep[...]) * ffb[...]
    h = _dot(jnp.maximum(agg, 0.0), pc[...]) + pcb[...]
    right = right_ref[...]
    h2 = jnp.maximum(_dot(h, o1a[...]) + _dot(right, o1b[...]) + o1bias[...],
                     0.0)
    node = _dot(h2, o2[...]) + o2b[...]
    if lt_ref is not None:
        lt_ref[...] = _dot(node, fr_next[...])
    oo_ref[...] = jnp.maximum(_dot(node, outw[...]) + outb[...], 0.0)


def _run_embed(cfp, vfp, mats):
    grid = N8 // BR8
    row_spec = pl.BlockSpec((BR8, 128), lambda i: (i, 0))
    mat_specs = [pl.BlockSpec(m.shape, lambda i: (0, 0)) for m in mats]
    out_sds = jax.ShapeDtypeStruct((N8, 128), jnp.float32)
    return pl.pallas_call(
        _tc_embed_kernel,
        grid=(grid,),
        in_specs=[row_spec, row_spec] + mat_specs,
        out_specs=[row_spec] * 5,
        out_shape=[out_sds] * 5,
    )(cfp, vfp, *mats)


def _run_post(s_parts, deg_parts, rightp, mats, with_lt):
    grid = N8 // BR8
    s_spec = pl.BlockSpec((NC, BR8, 128), lambda i: (0, i, 0))
    deg_spec = pl.BlockSpec((NC, BR8, 8), lambda i: (0, i, 0))
    row_spec = pl.BlockSpec((BR8, 128), lambda i: (i, 0))
    mat_specs = [pl.BlockSpec(m.shape, lambda i: (0, 0)) for m in mats]
    out_sds = jax.ShapeDtypeStruct((N8, 128), jnp.float32)
    if with_lt:
        kern = lambda *a: _tc_post_kernel(*a[:16], a[16], a[17])
        out_specs, out_shape = [row_spec] * 2, [out_sds] * 2
    else:
        kern = lambda *a: _tc_post_kernel(*a[:13], None, a[13], a[14], None,
                                          a[15])
        out_specs, out_shape = [row_spec], [out_sds]
    return pl.pallas_call(
        kern,
        grid=(grid,),
        in_specs=[s_spec, deg_spec, row_spec] + mat_specs,
        out_specs=out_specs,
        out_shape=out_shape,
    )(s_parts, deg_parts, rightp, *mats)


def kernel(constraint_features, edge_indices, edge_features, variable_features,
           params):
    p = params
    f32 = jnp.float32

    def pack_nodes(x):
        x = jnp.pad(x.astype(f32), ((0, NPAD - N_NODES), (0, 0)))
        return x.reshape(N8, 128)

    cfp = pack_nodes(constraint_features)
    vfp = pack_nodes(variable_features)

    e0 = edge_indices[0].astype(jnp.int32)
    e1 = edge_indices[1].astype(jnp.int32)
    pad_e = E_PAD - N_EDGES
    e0 = jnp.pad(e0, (0, pad_e), constant_values=N_NODES).reshape(-1, 128)
    e1 = jnp.pad(e1, (0, pad_e), constant_values=N_NODES).reshape(-1, 128)
    ef = jnp.pad(edge_features.astype(f32).reshape(-1), (0, pad_e))

    z2d = jnp.zeros((TSLICE, EMB), f32)
    z1d = jnp.zeros((TSLICE,), f32)

    def wrow(fe_w, eew, eeb):
        return jnp.stack([fe_w.astype(f32).reshape(EMB),
                          jnp.full((EMB,), eew, f32),
                          jnp.full((EMB,), eeb, f32),
                          jnp.zeros((EMB,), f32)])

    eew = p['ee_w'].reshape(())
    eeb = p['ee_b'].reshape(())
    w1 = wrow(p['cv']['fe_w'], eew, eeb)
    w2 = wrow(p['cc']['fe_w'], eew, eeb)

    rep = jnp.repeat(jnp.eye(8, dtype=f32), 16, axis=1)

    embed_mats = [
        _kron8(p['ce_w1']), _tile8(p['ce_b1']),
        _kron8(p['ce_w2']), _tile8(p['ce_b2']),
        _kron8(p['ve_w1']), _tile8(p['ve_b1']),
        _kron8(p['ve_w2']), _tile8(p['ve_b2']),
        _kron8(p['cv']['fl_w']), _tile8(p['cv']['fl_b']),
        _kron8(p['cv']['fr_w']),
        _kron8(p['cc']['fl_w']), _tile8(p['cc']['fl_b']),
    ]
    c0p, rt1p, v0p, lt1p, rt2p = _run_embed(cfp, vfp, embed_mats)

    deg_outs = _build_sc_deg()(e1, e0, z1d)
    dega = jnp.stack(deg_outs[:NC])
    degb = jnp.stack(deg_outs[NC:])
    deg1 = degb.reshape(NC, N8, 8)
    deg2 = dega.reshape(NC, N8, 8)

    sc_edge = _build_sc_pass()
    lt1 = lt1p.reshape(NPAD, EMB)
    rt1 = rt1p.reshape(NPAD, EMB)
    (s1_parts,) = sc_edge(e1, e0, ef, lt1, rt1, w1, z2d)
    s1 = s1_parts.reshape(NC, N8, 128)

    pcv = p['cv']
    post1_mats = [
        rep, _kron8(pcv['ff_w']), _tile8(pcv['ff_b']),
        _kron8(pcv['pc_w']), _tile8(pcv['pc_b']),
        _kron8(pcv['o1_w'][:EMB]), _kron8(pcv['o1_w'][EMB:]),
        _tile8(pcv['o1_b']),
        _kron8(pcv['o2_w']), _tile8(pcv['o2_b']),
        _kron8(p['cc']['fr_w']),
        _kron8(p['oc_w']), _tile8(p['oc_b']),
    ]
    lt2p, ocp = _run_post(s1, deg1, c0p, post1_mats, with_lt=True)

    lt2 = lt2p.reshape(NPAD, EMB)
    rt2 = rt2p.reshape(NPAD, EMB)
    (s2_parts,) = sc_edge(e0, e1, ef, lt2, rt2, w2, z2d)
    s2 = s2_parts.reshape(NC, N8, 128)

    pcc = p['cc']
    post2_mats = [
        rep, _kron8(pcc['ff_w']), _tile8(pcc['ff_b']),
        _kron8(pcc['pc_w']), _tile8(pcc['pc_b']),
        _kron8(pcc['o1_w'][:EMB]), _kron8(pcc['o1_w'][EMB:]),
        _tile8(pcc['o1_b']),
        _kron8(pcc['o2_w']), _tile8(pcc['o2_b']),
        _kron8(p['ov_w']), _tile8(p['ov_b']),
    ]
    (ovp,) = _run_post(s2, deg2, v0p, post2_mats, with_lt=False)

    out_var = ovp.reshape(NPAD, EMB)[:N_NODES]
    out_cons = ocp.reshape(NPAD, EMB)[:N_NODES]
    return (out_var, out_cons)


# speedup vs baseline: 1.0917x; 1.0156x over previous
import functools

import jax
import jax.numpy as jnp
from jax import lax
from jax.experimental import pallas as pl
from jax.experimental.pallas import tpu as pltpu
from jax.experimental.pallas import tpu_sc as plsc

EMB = 16
N_NODES = 100000
N_EDGES = 3200000

NC = 2
NS = 16
NW = NC * NS

NPAD = 100352
N8 = NPAD // 8
TSLICE = NPAD // NS

CHUNK_P = 256
CHUNK_D = 2048
E_PAD = NW * CHUNK_D * (-(-N_EDGES // (NW * CHUNK_D)))
EW = E_PAD // NW


def _gather_rows(table_hbm, idx_ref, dst_ref, sem):
    return pltpu.async_copy(table_hbm.at[idx_ref], dst_ref, sem)


def _gather_drain(table_hbm, idx_ref, dst_ref, sem):
    pltpu.make_async_copy(table_hbm.at[idx_ref], dst_ref, sem).wait()


def _scatter_add(src_ref, acc_ref, idx_ref):
    pltpu.sync_copy(src_ref, acc_ref.at[idx_ref], add=True)


def _gather16(ref, idx0):
    return plsc.load_gather(ref, [idx0])


def _edge_loop(n, fn, unroll):
    plsc.parallel_loop(0, n, 1, unroll=unroll)(fn)


def _axis_index(name):
    return lax.axis_index(name)


def _barrier():
    plsc.subcore_barrier()


def _sc_mesh():
    return plsc.VectorSubcoreMesh(core_axis_name="c", subcore_axis_name="s",
                                  num_cores=NC, num_subcores=NS)


_SC_PARAMS = dict(
    compiler_params=pltpu.CompilerParams(
        needs_layout_passes=False, use_tc_tiling_on_sc=False))


def _build_sc_deg():
    g_n = EW // CHUNK_D
    out_type = [jax.ShapeDtypeStruct((NPAD,), jnp.float32)
                for _ in range(2 * NC)]
    scratch = [
        pltpu.VMEM_SHARED((NPAD,), jnp.float32),
        pltpu.VMEM_SHARED((NPAD,), jnp.float32),
        pltpu.VMEM((CHUNK_D,), jnp.int32),
        pltpu.VMEM((CHUNK_D,), jnp.int32),
        pltpu.VMEM((CHUNK_D,), jnp.float32),
    ]

    def body(aidx_hbm, bidx_hbm, z1d_hbm, *rest):
        dega_outs = rest[:NC]
        degb_outs = rest[NC:2 * NC]
        dega_sh, degb_sh, aidx_v, bidx_v, ones_v = rest[2 * NC:]

        cid = _axis_index("c")
        sid = _axis_index("s")
        wid = cid * NS + sid
        toff = sid * TSLICE

        pltpu.sync_copy(z1d_hbm, dega_sh.at[pl.ds(toff, TSLICE)])
        pltpu.sync_copy(z1d_hbm, degb_sh.at[pl.ds(toff, TSLICE)])
        ones16 = jnp.full((16,), 1.0, jnp.float32)
        for t in range(CHUNK_D // 16):
            ones_v[pl.ds(16 * t, 16)] = ones16
        _barrier()

        eb0 = wid * EW

        def super_body(g, carry):
            eb = eb0 + g * CHUNK_D
            pltpu.sync_copy(aidx_hbm.at[pl.ds(eb, CHUNK_D)], aidx_v)
            pltpu.sync_copy(bidx_hbm.at[pl.ds(eb, CHUNK_D)], bidx_v)
            _scatter_add(ones_v, dega_sh, aidx_v)
            _scatter_add(ones_v, degb_sh, bidx_v)
            return carry

        lax.fori_loop(0, g_n, super_body, 0)
        _barrier()

        for k in range(NC):
            @pl.when(cid == k)
            def _copy_degs(k=k):
                pltpu.sync_copy(dega_sh.at[pl.ds(toff, TSLICE)],
                                dega_outs[k].at[pl.ds(toff, TSLICE)])
                pltpu.sync_copy(degb_sh.at[pl.ds(toff, TSLICE)],
                                degb_outs[k].at[pl.ds(toff, TSLICE)])

    return pl.kernel(body, out_type=out_type, mesh=_sc_mesh(),
                     scratch_types=scratch, **_SC_PARAMS)


def _build_sc_pass():
    sub = CHUNK_P // 128
    g_n = EW // CHUNK_P
    assert g_n % 2 == 0
    out_type = [jax.ShapeDtypeStruct((NC, NPAD, EMB), jnp.float32)]
    buf = lambda: [
        pltpu.VMEM((CHUNK_P,), jnp.int32),
        pltpu.VMEM((CHUNK_P,), jnp.int32),
        pltpu.VMEM((CHUNK_P,), jnp.float32),
        pltpu.VMEM((CHUNK_P, EMB), jnp.float32),
        pltpu.VMEM((CHUNK_P, EMB), jnp.float32),
        pltpu.SemaphoreType.DMA,
    ]
    scratch = (
        [pltpu.VMEM_SHARED((NPAD, EMB), jnp.float32)]
        + buf() + buf()
        + [pltpu.VMEM((4, EMB), jnp.float32)]
    )

    def body(aidx_hbm, bidx_hbm, ef_hbm, left_hbm, right_hbm, w_hbm,
             z2d_hbm, s_out, s_sh, *rest):
        bufs = [rest[0:6], rest[6:12]]
        w_v = rest[12]
        cid = _axis_index("c")
        sid = _axis_index("s")
        wid = cid * NS + sid
        toff = sid * TSLICE

        pltpu.sync_copy(z2d_hbm, s_sh.at[pl.ds(toff, TSLICE), :])
        pltpu.sync_copy(w_hbm, w_v)
        _barrier()

        fe = w_v[0]
        eew = w_v[1]
        eeb = w_v[2]
        zero16 = jnp.zeros((16,), jnp.float32)

        eb0 = wid * EW

        def load_and_fire(g, b):
            aidx_v, bidx_v, ef_v, l_v, r_v, sem = bufs[b]
            eb = eb0 + g * CHUNK_P
            pltpu.sync_copy(aidx_hbm.at[pl.ds(eb, CHUNK_P)], aidx_v)
            pltpu.sync_copy(bidx_hbm.at[pl.ds(eb, CHUNK_P)], bidx_v)
            pltpu.sync_copy(ef_hbm.at[pl.ds(eb, CHUNK_P)], ef_v)
            _gather_rows(left_hbm, aidx_v, l_v, sem)
            _gather_rows(right_hbm, bidx_v, r_v, sem)
            for t in range(CHUNK_P // 16):
                s = pl.ds(16 * t, 16)
                ef_v[s] = jnp.maximum(ef_v[s] * eew + eeb, zero16)

        def drain_compute_scatter(b):
            aidx_v, bidx_v, ef_v, l_v, r_v, sem = bufs[b]
            _gather_drain(left_hbm, aidx_v, l_v, sem)
            _gather_drain(right_hbm, bidx_v, r_v, sem)

            def edge_body(j):
                bj = jnp.broadcast_to(j, (16,))
                e16 = _gather16(ef_v, bj)
                l_v[j] = jnp.maximum(l_v[j] + r_v[j] + e16 * fe, zero16)
            _edge_loop(CHUNK_P, edge_body, 8)

            _scatter_add(l_v, s_sh, bidx_v)

        load_and_fire(0, 0)

        def super_body(h, carry):
            g = 2 * h
            load_and_fire(g + 1, 1)
            drain_compute_scatter(0)

            @pl.when(h < g_n // 2 - 1)
            def _prefetch():
                load_and_fire(g + 2, 0)
            drain_compute_scatter(1)
            return carry

        lax.fori_loop(0, g_n // 2, super_body, 0)
        _barrier()

        pltpu.sync_copy(s_sh.at[pl.ds(toff, TSLICE), :],
                        s_out.at[cid, pl.ds(toff, TSLICE), :])

    return pl.kernel(body, out_type=out_type, mesh=_sc_mesh(),
                     scratch_types=scratch, **_SC_PARAMS)


BR8 = 3136


def _kron8(w):
    return jnp.kron(jnp.eye(8, dtype=jnp.float32), w.astype(jnp.float32))


def _tile8(b):
    return jnp.tile(b.astype(jnp.float32), 8).reshape(1, 128)


def _dot(x, w):
    return jnp.dot(x, w, preferred_element_type=jnp.float32,
                   precision=jax.lax.Precision.HIGHEST)


def _tc_embed_kernel(cf_ref, vf_ref, ce1, ceb1, ce2, ceb2, ve1, veb1, ve2,
                     veb2, fl1, flb1, fr1, fl2, flb2,
                     c0_ref, rt1_ref, v0_ref, lt1_ref, rt2_ref):
    c = jnp.maximum(_dot(cf_ref[...], ce1[...]) + ceb1[...], 0.0)
    c = jnp.maximum(_dot(c, ce2[...]) + ceb2[...], 0.0)
    c0_ref[...] = c
    rt1_ref[...] = _dot(c, fl1[...]) + flb1[...]
    v = jnp.maximum(_dot(vf_ref[...], ve1[...]) + veb1[...], 0.0)
    v = jnp.maximum(_dot(v, ve2[...]) + veb2[...], 0.0)
    v0_ref[...] = v
    lt1_ref[...] = _dot(v, fr1[...])
    rt2_ref[...] = _dot(v, fl2[...]) + flb2[...]


def _tc_post_kernel(s_ref, deg_ref, right_ref, rep, ff, ffb, pc, pcb, o1a,
                    o1b, o1bias, o2, o2b, fr_next, outw, outb,
                    lt_ref, oo_ref):
    s = jnp.sum(s_ref[...], axis=0)
    deg = jnp.sum(deg_ref[...], axis=0)
    agg = _dot(s, ff[...]) + _dot(deg, rep[...]) * ffb[...]
    h = _dot(jnp.maximum(agg, 0.0), pc[...]) + pcb[...]
    right = right_ref[...]
    h2 = jnp.maximum(_dot(h, o1a[...]) + _dot(right, o1b[...]) + o1bias[...],
                     0.0)
    node = _dot(h2, o2[...]) + o2b[...]
    if lt_ref is not None:
        lt_ref[...] = _dot(node, fr_next[...])
    oo_ref[...] = jnp.maximum(_dot(node, outw[...]) + outb[...], 0.0)


def _run_embed(cfp, vfp, mats):
    grid = N8 // BR8
    row_spec = pl.BlockSpec((BR8, 128), lambda i: (i, 0))
    mat_specs = [pl.BlockSpec(m.shape, lambda i: (0, 0)) for m in mats]
    out_sds = jax.ShapeDtypeStruct((N8, 128), jnp.float32)
    return pl.pallas_call(
        _tc_embed_kernel,
        grid=(grid,),
        in_specs=[row_spec, row_spec] + mat_specs,
        out_specs=[row_spec] * 5,
        out_shape=[out_sds] * 5,
    )(cfp, vfp, *mats)


def _run_post(s_parts, deg_parts, rightp, mats, with_lt):
    grid = N8 // BR8
    s_spec = pl.BlockSpec((NC, BR8, 128), lambda i: (0, i, 0))
    deg_spec = pl.BlockSpec((NC, BR8, 8), lambda i: (0, i, 0))
    row_spec = pl.BlockSpec((BR8, 128), lambda i: (i, 0))
    mat_specs = [pl.BlockSpec(m.shape, lambda i: (0, 0)) for m in mats]
    out_sds = jax.ShapeDtypeStruct((N8, 128), jnp.float32)
    if with_lt:
        kern = lambda *a: _tc_post_kernel(*a[:16], a[16], a[17])
        out_specs, out_shape = [row_spec] * 2, [out_sds] * 2
    else:
        kern = lambda *a: _tc_post_kernel(*a[:13], None, a[13], a[14], None,
                                          a[15])
        out_specs, out_shape = [row_spec], [out_sds]
    return pl.pallas_call(
        kern,
        grid=(grid,),
        in_specs=[s_spec, deg_spec, row_spec] + mat_specs,
        out_specs=out_specs,
        out_shape=out_shape,
    )(s_parts, deg_parts, rightp, *mats)


def kernel(constraint_features, edge_indices, edge_features, variable_features,
           params):
    p = params
    f32 = jnp.float32

    def pack_nodes(x):
        x = jnp.pad(x.astype(f32), ((0, NPAD - N_NODES), (0, 0)))
        return x.reshape(N8, 128)

    cfp = pack_nodes(constraint_features)
    vfp = pack_nodes(variable_features)

    e0 = edge_indices[0].astype(jnp.int32)
    e1 = edge_indices[1].astype(jnp.int32)
    pad_e = E_PAD - N_EDGES
    e0 = jnp.pad(e0, (0, pad_e), constant_values=N_NODES)
    e1 = jnp.pad(e1, (0, pad_e), constant_values=N_NODES)
    ef = jnp.pad(edge_features.astype(f32).reshape(-1), (0, pad_e))

    z2d = jnp.zeros((TSLICE, EMB), f32)
    z1d = jnp.zeros((TSLICE,), f32)

    def wrow(fe_w, eew, eeb):
        return jnp.stack([fe_w.astype(f32).reshape(EMB),
                          jnp.full((EMB,), eew, f32),
                          jnp.full((EMB,), eeb, f32),
                          jnp.zeros((EMB,), f32)])

    eew = p['ee_w'].reshape(())
    eeb = p['ee_b'].reshape(())
    w1 = wrow(p['cv']['fe_w'], eew, eeb)
    w2 = wrow(p['cc']['fe_w'], eew, eeb)

    rep = jnp.repeat(jnp.eye(8, dtype=f32), 16, axis=1)

    embed_mats = [
        _kron8(p['ce_w1']), _tile8(p['ce_b1']),
        _kron8(p['ce_w2']), _tile8(p['ce_b2']),
        _kron8(p['ve_w1']), _tile8(p['ve_b1']),
        _kron8(p['ve_w2']), _tile8(p['ve_b2']),
        _kron8(p['cv']['fl_w']), _tile8(p['cv']['fl_b']),
        _kron8(p['cv']['fr_w']),
        _kron8(p['cc']['fl_w']), _tile8(p['cc']['fl_b']),
    ]
    c0p, rt1p, v0p, lt1p, rt2p = _run_embed(cfp, vfp, embed_mats)

    deg_outs = _build_sc_deg()(e1, e0, z1d)
    dega = jnp.stack(deg_outs[:NC])
    degb = jnp.stack(deg_outs[NC:])
    deg1 = degb.reshape(NC, N8, 8)
    deg2 = dega.reshape(NC, N8, 8)

    sc_edge = _build_sc_pass()
    lt1 = lt1p.reshape(NPAD, EMB)
    rt1 = rt1p.reshape(NPAD, EMB)
    (s1_parts,) = sc_edge(e1, e0, ef, lt1, rt1, w1, z2d)
    s1 = s1_parts.reshape(NC, N8, 128)

    pcv = p['cv']
    post1_mats = [
        rep, _kron8(pcv['ff_w']), _tile8(pcv['ff_b']),
        _kron8(pcv['pc_w']), _tile8(pcv['pc_b']),
        _kron8(pcv['o1_w'][:EMB]), _kron8(pcv['o1_w'][EMB:]),
        _tile8(pcv['o1_b']),
        _kron8(pcv['o2_w']), _tile8(pcv['o2_b']),
        _kron8(p['cc']['fr_w']),
        _kron8(p['oc_w']), _tile8(p['oc_b']),
    ]
    lt2p, ocp = _run_post(s1, deg1, c0p, post1_mats, with_lt=True)

    lt2 = lt2p.reshape(NPAD, EMB)
    rt2 = rt2p.reshape(NPAD, EMB)
    (s2_parts,) = sc_edge(e0, e1, ef, lt2, rt2, w2, z2d)
    s2 = s2_parts.reshape(NC, N8, 128)

    pcc = p['cc']
    post2_mats = [
        rep, _kron8(pcc['ff_w']), _tile8(pcc['ff_b']),
        _kron8(pcc['pc_w']), _tile8(pcc['pc_b']),
        _kron8(pcc['o1_w'][:EMB]), _kron8(pcc['o1_w'][EMB:]),
        _tile8(pcc['o1_b']),
        _kron8(pcc['o2_w']), _tile8(pcc['o2_b']),
        _kron8(p['ov_w']), _tile8(p['ov_b']),
    ]
    (ovp,) = _run_post(s2, deg2, v0p, post2_mats, with_lt=False)

    out_var = ovp.reshape(NPAD, EMB)[:N_NODES]
    out_cons = ocp.reshape(NPAD, EMB)[:N_NODES]
    return (out_var, out_cons)
